# baseline (speedup 1.0000x reference)
import functools

import jax
import jax.numpy as jnp
from jax import lax
from jax.experimental import pallas as pl
from jax.experimental.pallas import tpu as pltpu
from jax.experimental.pallas import tpu_sc as plsc

_MAX_YR_LEN = 106
_T = _MAX_YR_LEN * 12
_B = 16384
_NC, _NS = 2, 16
_NW = _NC * _NS
_COLS_PER_W = _B // _NW
_L = 16
_CG = _COLS_PER_W // 128
_TB = 424
_NTB = _T // _TB


@functools.partial(
    pl.kernel,
    out_type=jax.ShapeDtypeStruct((_T, _B), jnp.float32),
    mesh=plsc.VectorSubcoreMesh(core_axis_name="c", subcore_axis_name="s"),
    compiler_params=pltpu.CompilerParams(needs_layout_passes=False),
    scratch_types=[
        pltpu.VMEM((4, _T), jnp.float32),
        pltpu.VMEM((_COLS_PER_W,), jnp.int32),
        pltpu.VMEM((2, _TB, 128), jnp.float32),
        pltpu.SemaphoreType.DMA,
        pltpu.SemaphoreType.DMA,
    ],
)
def _sc_lookup_t(table_hbm, idx_hbm, out_hbm, table_v, idx_v, buf_v,
                 sem0, sem1):
    wid = lax.axis_index("s") * _NC + lax.axis_index("c")
    base_b = wid * _COLS_PER_W
    pltpu.sync_copy(table_hbm, table_v)
    pltpu.sync_copy(idx_hbm.at[pl.ds(base_b, _COLS_PER_W)], idx_v)
    sems = (sem0, sem1)

    def _out_dma(blk, slot):
        cg, tb = blk // _NTB, blk % _NTB
        return pltpu.make_async_copy(
            buf_v.at[slot],
            out_hbm.at[pl.ds(tb * _TB, _TB),
                       pl.ds(base_b + cg * 128, 128)],
            sems[slot])

    for blk in range(_CG * _NTB):
        slot = blk % 2
        cg, tb = blk // _NTB, blk % _NTB
        if blk >= 2:
            _out_dma(blk - 2, slot).wait()
        cvecs = [idx_v[pl.ds(cg * 128 + g * _L, _L)] for g in range(8)]

        @pl.loop(0, _TB, unroll=2)
        def _fill(t_local, _cvecs=cvecs, _tb=tb, _slot=slot):
            t = jnp.int32(_tb * _TB) + t_local
            tvec = jnp.full((_L,), t, jnp.int32)
            for g in range(8):
                val = plsc.load_gather(table_v, [_cvecs[g], tvec])
                buf_v[_slot, t_local, pl.ds(g * _L, _L)] = val

        _out_dma(blk, slot).start()

    _out_dma(_CG * _NTB - 2, 0).wait()
    _out_dma(_CG * _NTB - 1, 1).wait()


def kernel(mp_idx, mp_val, qx):
    del mp_val
    qm = jnp.power(qx + 1.0, 1.0 / 12.0) - 1.0
    rep = jnp.repeat(qm, 12, axis=1)
    sh = jnp.concatenate(
        [rep[:, 12:], jnp.zeros((2, 12), rep.dtype)], axis=1)
    table = jnp.stack([rep[0], sh[0], rep[1], sh[1]], axis=0)
    idx = (mp_idx[:, 0].astype(jnp.int32) * 2
           + mp_idx[:, 1].astype(jnp.int32))
    return _sc_lookup_t(table, idx).T


# speedup vs baseline: 2.6719x; 2.6719x over previous
import functools

import jax
import jax.numpy as jnp
from jax import lax
from jax.experimental import pallas as pl
from jax.experimental.pallas import tpu as pltpu
from jax.experimental.pallas import tpu_sc as plsc

_MAX_YR_LEN = 106
_T = _MAX_YR_LEN * 12
_B = 16384
_NC, _NS = 2, 16
_NW = _NC * _NS
_COLS_PER_W = _B // _NW
_L = 16
_CG = _COLS_PER_W // 128
_TB = 424
_NTB = _T // _TB


@functools.partial(
    pl.kernel,
    out_type=jax.ShapeDtypeStruct((_T, _B), jnp.float32),
    mesh=plsc.VectorSubcoreMesh(core_axis_name="c", subcore_axis_name="s"),
    compiler_params=pltpu.CompilerParams(
        needs_layout_passes=False, disable_bounds_checks=True),
    scratch_types=[
        pltpu.VMEM((4 * _T,), jnp.float32),
        pltpu.VMEM((_COLS_PER_W,), jnp.int32),
        pltpu.VMEM((2, _TB, 128), jnp.float32),
        pltpu.SemaphoreType.DMA,
        pltpu.SemaphoreType.DMA,
    ],
)
def _sc_lookup_t(table_hbm, idx_hbm, out_hbm, table_v, idx_v, buf_v,
                 sem0, sem1):
    wid = lax.axis_index("s") * _NC + lax.axis_index("c")
    base_b = wid * _COLS_PER_W
    pltpu.sync_copy(table_hbm, table_v)
    pltpu.sync_copy(idx_hbm.at[pl.ds(base_b, _COLS_PER_W)], idx_v)
    sems = (sem0, sem1)

    def _out_dma(blk, slot):
        cg, tb = blk // _NTB, blk % _NTB
        return pltpu.make_async_copy(
            buf_v.at[slot],
            out_hbm.at[pl.ds(tb * _TB, _TB),
                       pl.ds(base_b + cg * 128, 128)],
            sems[slot])

    for blk in range(_CG * _NTB):
        slot = blk % 2
        cg, tb = blk // _NTB, blk % _NTB
        if blk >= 2:
            _out_dma(blk - 2, slot).wait()
        ivecs0 = tuple(
            idx_v[pl.ds(cg * 128 + g * _L, _L)] * jnp.int32(_T)
            + jnp.int32(tb * _TB)
            for g in range(8))

        @pl.loop(0, _TB, init_carry=ivecs0, unroll=4)
        def _fill(t_local, ivecs, _slot=slot):
            for g in range(8):
                val = plsc.load_gather(table_v, [ivecs[g]])
                buf_v[_slot, t_local, pl.ds(g * _L, _L)] = val
            return tuple(iv + jnp.int32(1) for iv in ivecs)

        _out_dma(blk, slot).start()

    _out_dma(_CG * _NTB - 2, 0).wait()
    _out_dma(_CG * _NTB - 1, 1).wait()


def kernel(mp_idx, mp_val, qx):
    del mp_val
    qm = jnp.power(qx + 1.0, 1.0 / 12.0) - 1.0
    rep = jnp.repeat(qm, 12, axis=1)
    sh = jnp.concatenate(
        [rep[:, 12:], jnp.zeros((2, 12), rep.dtype)], axis=1)
    table = jnp.stack([rep[0], sh[0], rep[1], sh[1]], axis=0)
    idx = (mp_idx[:, 0].astype(jnp.int32) * 2
           + mp_idx[:, 1].astype(jnp.int32))
    return _sc_lookup_t(table.reshape(4 * _T), idx).T


# speedup vs baseline: 9.2490x; 3.4616x over previous
import functools

import jax
import jax.numpy as jnp
from jax import lax
from jax.experimental import pallas as pl
from jax.experimental.pallas import tpu as pltpu
from jax.experimental.pallas import tpu_sc as plsc

_MAX_YR_LEN = 106
_T = _MAX_YR_LEN * 12
_B = 16384
_NC, _NS = 2, 16
_NW = _NC * _NS
_COLS_PER_W = _B // _NW
_L = 16
_CG = _COLS_PER_W // 128
_TB = 424
_NTB = _T // _TB


@functools.partial(
    pl.kernel,
    out_type=jax.ShapeDtypeStruct((_T, _B), jnp.float32),
    mesh=plsc.VectorSubcoreMesh(core_axis_name="c", subcore_axis_name="s"),
    compiler_params=pltpu.CompilerParams(
        needs_layout_passes=False, disable_bounds_checks=True),
    scratch_types=[
        pltpu.VMEM((4 * _T,), jnp.float32),
        pltpu.VMEM((_COLS_PER_W,), jnp.int32),
        pltpu.VMEM((2, _TB, 128), jnp.float32),
        pltpu.SemaphoreType.DMA,
        pltpu.SemaphoreType.DMA,
    ],
)
def _sc_lookup_t(table_hbm, idx_hbm, out_hbm, table_v, idx_v, buf_v,
                 sem0, sem1):
    wid = lax.axis_index("s") * _NC + lax.axis_index("c")
    base_b = wid * _COLS_PER_W
    pltpu.sync_copy(table_hbm, table_v)
    pltpu.sync_copy(idx_hbm.at[pl.ds(base_b, _COLS_PER_W)], idx_v)
    sems = (sem0, sem1)

    def _out_dma(blk, slot):
        cg, tb = blk // _NTB, blk % _NTB
        return pltpu.make_async_copy(
            buf_v.at[slot],
            out_hbm.at[pl.ds(tb * _TB, _TB),
                       pl.ds(base_b + cg * 128, 128)],
            sems[slot])

    for blk in range(_CG * _NTB):
        slot = blk % 2
        cg, tb = blk // _NTB, blk % _NTB
        if blk >= 2:
            _out_dma(blk - 2, slot).wait()
        ivecs0 = tuple(
            idx_v[pl.ds(cg * 128 + g * _L, _L)] * jnp.int32(_T)
            + jnp.int32(tb * _TB)
            for g in range(8))

        @plsc.parallel_loop(0, _TB, unroll=8, carry=ivecs0)
        def _fill(t_local, ivecs, _slot=slot):
            for g in range(8):
                val = plsc.load_gather(table_v, [ivecs[g]])
                buf_v[_slot, t_local, pl.ds(g * _L, _L)] = val
            return tuple(iv + jnp.int32(1) for iv in ivecs)

        _out_dma(blk, slot).start()

    _out_dma(_CG * _NTB - 2, 0).wait()
    _out_dma(_CG * _NTB - 1, 1).wait()


def kernel(mp_idx, mp_val, qx):
    del mp_val
    qm = jnp.power(qx + 1.0, 1.0 / 12.0) - 1.0
    rep = jnp.repeat(qm, 12, axis=1)
    sh = jnp.concatenate(
        [rep[:, 12:], jnp.zeros((2, 12), rep.dtype)], axis=1)
    table = jnp.stack([rep[0], sh[0], rep[1], sh[1]], axis=0)
    idx = (mp_idx[:, 0].astype(jnp.int32) * 2
           + mp_idx[:, 1].astype(jnp.int32))
    return _sc_lookup_t(table.reshape(4 * _T), idx).T
